# Initial kernel scaffold; baseline (speedup 1.0000x reference)
import jax
import jax.numpy as jnp
from jax.experimental import pallas as pl


def kernel(x, edge_index, conv_w, conv_b, lin1_w, lin1_b, lin2_w, lin2_b, lin3_w, lin3_b, deterministic):
    raise NotImplementedError("write your pallas kernel here")


# speedup vs baseline: 14.9240x; 14.9240x over previous
import functools

import jax
import jax.numpy as jnp
from jax import lax
from jax.experimental import pallas as pl
from jax.experimental.pallas import tpu as pltpu
from jax.experimental.pallas import tpu_sc as plsc

N_NODES = 10000
N_EDGES = 320000
EROWS = N_EDGES // 128
F = 64
C_IN = 128
HID = 256
NC, NS = 2, 16
NPAD = 10240
RPT = NPAD // NS


def _mesh():
    return plsc.VectorSubcoreMesh(core_axis_name="c", subcore_axis_name="s")


def _deg_body(dst_ref, ones_ref, zeros_ref, out_ref, idx_d, ones_v, acc):
    cid = lax.axis_index("c")
    sid = lax.axis_index("s")
    r0 = sid * RPT
    pltpu.sync_copy(zeros_ref.at[pl.ds(r0, RPT)], acc.at[pl.ds(r0, RPT)])
    pltpu.sync_copy(ones_ref, ones_v)
    plsc.subcore_barrier()
    half = EROWS // NC
    base = cid * half
    iters = (half + NS - 1) // NS

    def body(rr, carry):
        r = base + sid + rr * NS

        @pl.when(r < base + half)
        def _go():
            pltpu.sync_copy(dst_ref.at[pl.ds(r * 128, 128)], idx_d)
            pltpu.sync_copy(ones_v, acc.at[idx_d], add=True)

        return carry

    lax.fori_loop(0, iters, body, None)
    plsc.subcore_barrier()
    pltpu.sync_copy(acc.at[pl.ds(r0, RPT)],
                    out_ref.at[pl.ds(cid * NPAD + r0, RPT)])


_deg_call = pl.kernel(
    _deg_body,
    out_type=jax.ShapeDtypeStruct((NC * NPAD, 128), jnp.float32),
    mesh=_mesh(),
    scratch_types=[
        pltpu.VMEM((128,), jnp.int32),
        pltpu.VMEM((128, 128), jnp.float32),
        pltpu.VMEM_SHARED((NPAD, 128), jnp.float32),
    ],
)


def _agg_body(src_ref, dst_ref, g_ref, zeros_ref, out_ref,
              idx_s, idx_d, rows_v, acc, sem):
    cid = lax.axis_index("c")
    sid = lax.axis_index("s")
    r0 = sid * RPT
    pltpu.sync_copy(zeros_ref.at[pl.ds(r0, RPT)], acc.at[pl.ds(r0, RPT)])
    plsc.subcore_barrier()
    half = EROWS // NC
    base = cid * half
    iters = (half + NS - 1) // NS

    def body(rr, carry):
        r = base + sid + rr * NS

        @pl.when(r < base + half)
        def _go():
            pltpu.sync_copy(src_ref.at[pl.ds(r * 128, 128)], idx_s)
            pltpu.sync_copy(dst_ref.at[pl.ds(r * 128, 128)], idx_d)
            pltpu.async_copy(g_ref.at[idx_s], rows_v, sem).wait()
            pltpu.sync_copy(rows_v, acc.at[idx_d], add=True)

        return carry

    lax.fori_loop(0, iters, body, None)
    plsc.subcore_barrier()
    pltpu.sync_copy(acc.at[pl.ds(r0, RPT)],
                    out_ref.at[pl.ds(cid * NPAD + r0, RPT)])


_agg_call = pl.kernel(
    _agg_body,
    out_type=jax.ShapeDtypeStruct((NC * NPAD, C_IN), jnp.float32),
    mesh=_mesh(),
    scratch_types=[
        pltpu.VMEM((128,), jnp.int32),
        pltpu.VMEM((128,), jnp.int32),
        pltpu.VMEM((128, C_IN), jnp.float32),
        pltpu.VMEM_SHARED((NPAD, C_IN), jnp.float32),
        pltpu.SemaphoreType.DMA,
    ],
)


_RB = 1000
_NB = N_NODES // _RB


def _gscale_body(x_ref, w_ref, deg_ref, g_ref):
    h = jnp.dot(x_ref[...], w_ref[...], preferred_element_type=jnp.float32,
                precision=lax.Precision.HIGHEST)
    deg = deg_ref[0] + deg_ref[1] + 1.0
    g_ref[...] = h * lax.rsqrt(deg)


def _gscale_call(x, conv_w, degp):
    return pl.pallas_call(
        _gscale_body,
        grid=(_NB,),
        in_specs=[
            pl.BlockSpec((_RB, C_IN), lambda j: (j, 0)),
            pl.BlockSpec((C_IN, C_IN), lambda j: (0, 0)),
            pl.BlockSpec((NC, _RB, 1), lambda j: (0, j, 0)),
        ],
        out_specs=pl.BlockSpec((_RB, C_IN), lambda j: (j, 0)),
        out_shape=jax.ShapeDtypeStruct((N_NODES, C_IN), jnp.float32),
    )(x, conv_w, degp)


def _head_body(p_ref, g_ref, x_ref, deg_ref, cb_ref,
               w1_ref, b1_ref, w2_ref, b2_ref, w3_ref, b3_ref, out_ref):
    p = p_ref[0] + p_ref[1]
    g = g_ref[...]
    deg = deg_ref[0] + deg_ref[1] + 1.0
    dis = lax.rsqrt(deg)
    conv = (p + g) * dis + cb_ref[...]
    h2 = jnp.maximum(conv, 0.0) + x_ref[...]
    m1 = jnp.dot(h2, w1_ref[...], preferred_element_type=jnp.float32,
                 precision=lax.Precision.HIGHEST) + b1_ref[...]
    m1 = jnp.where(m1 > 0, m1, 0.01 * m1)
    m2 = jnp.dot(m1, w2_ref[...], preferred_element_type=jnp.float32,
                 precision=lax.Precision.HIGHEST) + b2_ref[...]
    m2 = jnp.where(m2 > 0, m2, 0.01 * m2)
    z = jnp.dot(m2, w3_ref[...], preferred_element_type=jnp.float32,
                precision=lax.Precision.HIGHEST) + b3_ref[...]
    out_ref[...] = jnp.maximum(z, 0.0) + jnp.log1p(jnp.exp(-jnp.abs(z)))


def _head_call(p, g, x, degp, cb, w1, b1, w2, b2, w3, b3):
    full = lambda shape: pl.BlockSpec(shape, lambda j: tuple(0 for _ in shape))
    return pl.pallas_call(
        _head_body,
        grid=(_NB,),
        in_specs=[
            pl.BlockSpec((NC, _RB, C_IN), lambda j: (0, j, 0)),
            pl.BlockSpec((_RB, C_IN), lambda j: (j, 0)),
            pl.BlockSpec((_RB, C_IN), lambda j: (j, 0)),
            pl.BlockSpec((NC, _RB, 1), lambda j: (0, j, 0)),
            full((1, C_IN)),
            full((C_IN, HID)),
            full((1, HID)),
            full((HID, HID)),
            full((1, HID)),
            full((HID, 1)),
            full((1, 1)),
        ],
        out_specs=pl.BlockSpec((_RB, 1), lambda j: (j, 0)),
        out_shape=jax.ShapeDtypeStruct((N_NODES, 1), jnp.float32),
    )(p, g, x, degp, cb, w1, b1, w2, b2, w3, b3)


def _norm_body(c_ref, a_ref):
    c = c_ref[...]
    a_ref[...] = c / (jnp.sum(c) + 1e-5)


def _norm_call(conc):
    return pl.pallas_call(
        _norm_body,
        out_shape=jax.ShapeDtypeStruct((1, N_NODES), jnp.float32),
    )(conc)


def kernel(x, edge_index, conv_w, conv_b, lin1_w, lin1_b, lin2_w, lin2_b,
           lin3_w, lin3_b, deterministic):
    src = edge_index[0]
    dst = edge_index[1]
    ones128 = jnp.ones((128, 128), jnp.float32)
    zerosw = jnp.zeros((NPAD, C_IN), jnp.float32)

    degp = (_deg_call(dst, ones128, zerosw)
            .reshape(NC, NPAD, 128)[:, :N_NODES, 0:1])
    g = _gscale_call(x, conv_w, degp)
    pflat = _agg_call(src, dst, g, zerosw)
    p = pflat.reshape(NC, NPAD, C_IN)[:, :N_NODES]
    conc = _head_call(p, g, x, degp, conv_b.reshape(1, C_IN),
                      lin1_w, lin1_b.reshape(1, HID),
                      lin2_w, lin2_b.reshape(1, HID),
                      lin3_w, lin3_b.reshape(1, 1))
    conc2 = conc.reshape(1, N_NODES)
    action = _norm_call(conc2)
    return action.reshape(N_NODES), conc2


# speedup vs baseline: 16.9201x; 1.1338x over previous
import functools

import jax
import jax.numpy as jnp
from jax import lax
from jax.experimental import pallas as pl
from jax.experimental.pallas import tpu as pltpu
from jax.experimental.pallas import tpu_sc as plsc

N_NODES = 10000
N_EDGES = 320000
EROWS = N_EDGES // 128
F = 64
C_IN = 128
HID = 256
NC, NS = 2, 16
NPAD = 10240
RPT = NPAD // NS


def _mesh():
    return plsc.VectorSubcoreMesh(core_axis_name="c", subcore_axis_name="s")


def _deg_body(dst_ref, ones_ref, zeros_ref, out_ref, idx_d, ones_v, acc):
    cid = lax.axis_index("c")
    sid = lax.axis_index("s")
    r0 = sid * RPT
    pltpu.sync_copy(zeros_ref.at[pl.ds(r0, RPT)], acc.at[pl.ds(r0, RPT)])
    pltpu.sync_copy(ones_ref, ones_v)
    plsc.subcore_barrier()
    half = EROWS // NC
    base = cid * half
    iters = (half + NS - 1) // NS

    def body(rr, carry):
        r = base + sid + rr * NS

        @pl.when(r < base + half)
        def _go():
            pltpu.sync_copy(dst_ref.at[pl.ds(r * 128, 128)], idx_d)
            pltpu.sync_copy(ones_v, acc.at[idx_d], add=True)

        return carry

    lax.fori_loop(0, iters, body, None)
    plsc.subcore_barrier()
    pltpu.sync_copy(acc.at[pl.ds(r0, RPT)],
                    out_ref.at[pl.ds(cid * NPAD + r0, RPT)])


_deg_call = pl.kernel(
    _deg_body,
    out_type=jax.ShapeDtypeStruct((NC * NPAD,), jnp.float32),
    mesh=_mesh(),
    scratch_types=[
        pltpu.VMEM((128,), jnp.int32),
        pltpu.VMEM((128,), jnp.float32),
        pltpu.VMEM_SHARED((NPAD,), jnp.float32),
    ],
)


def _agg_body(src_ref, dst_ref, g_ref, zeros_ref, out_ref,
              idx_s, idx_d, rows_v, acc, sem):
    cid = lax.axis_index("c")
    sid = lax.axis_index("s")
    r0 = sid * RPT
    pltpu.sync_copy(zeros_ref.at[pl.ds(r0, RPT)], acc.at[pl.ds(r0, RPT)])
    plsc.subcore_barrier()
    half = EROWS // NC
    base = cid * half
    iters = (half + NS - 1) // NS

    def body(rr, carry):
        r = base + sid + rr * NS

        @pl.when(r < base + half)
        def _go():
            pltpu.sync_copy(src_ref.at[pl.ds(r * 128, 128)], idx_s)
            pltpu.sync_copy(dst_ref.at[pl.ds(r * 128, 128)], idx_d)
            pltpu.async_copy(g_ref.at[idx_s], rows_v, sem).wait()
            pltpu.sync_copy(rows_v, acc.at[idx_d], add=True)

        return carry

    lax.fori_loop(0, iters, body, None)
    plsc.subcore_barrier()
    pltpu.sync_copy(acc.at[pl.ds(r0, RPT)],
                    out_ref.at[pl.ds(cid * NPAD + r0, RPT)])


_agg_call = pl.kernel(
    _agg_body,
    out_type=jax.ShapeDtypeStruct((NC * NPAD, C_IN), jnp.float32),
    mesh=_mesh(),
    scratch_types=[
        pltpu.VMEM((128,), jnp.int32),
        pltpu.VMEM((128,), jnp.int32),
        pltpu.VMEM((128, C_IN), jnp.float32),
        pltpu.VMEM_SHARED((NPAD, C_IN), jnp.float32),
        pltpu.SemaphoreType.DMA,
    ],
)


_RB = 1000
_NB = N_NODES // _RB


def _gscale_body(x_ref, w_ref, deg_ref, g_ref):
    h = jnp.dot(x_ref[...], w_ref[...], preferred_element_type=jnp.float32,
                precision=lax.Precision.HIGHEST)
    deg = deg_ref[0] + deg_ref[1] + 1.0
    g_ref[...] = h * lax.rsqrt(deg)


def _gscale_call(x, conv_w, degp):
    return pl.pallas_call(
        _gscale_body,
        grid=(_NB,),
        in_specs=[
            pl.BlockSpec((_RB, C_IN), lambda j: (j, 0)),
            pl.BlockSpec((C_IN, C_IN), lambda j: (0, 0)),
            pl.BlockSpec((NC, _RB, 1), lambda j: (0, j, 0)),
        ],
        out_specs=pl.BlockSpec((_RB, C_IN), lambda j: (j, 0)),
        out_shape=jax.ShapeDtypeStruct((N_NODES, C_IN), jnp.float32),
    )(x, conv_w, degp)


def _head_body(p_ref, g_ref, x_ref, deg_ref, cb_ref,
               w1_ref, b1_ref, w2_ref, b2_ref, w3_ref, b3_ref, out_ref):
    p = p_ref[0] + p_ref[1]
    g = g_ref[...]
    deg = deg_ref[0] + deg_ref[1] + 1.0
    dis = lax.rsqrt(deg)
    conv = (p + g) * dis + cb_ref[...]
    h2 = jnp.maximum(conv, 0.0) + x_ref[...]
    m1 = jnp.dot(h2, w1_ref[...], preferred_element_type=jnp.float32,
                 precision=lax.Precision.HIGHEST) + b1_ref[...]
    m1 = jnp.where(m1 > 0, m1, 0.01 * m1)
    m2 = jnp.dot(m1, w2_ref[...], preferred_element_type=jnp.float32,
                 precision=lax.Precision.HIGHEST) + b2_ref[...]
    m2 = jnp.where(m2 > 0, m2, 0.01 * m2)
    z = jnp.dot(m2, w3_ref[...], preferred_element_type=jnp.float32,
                precision=lax.Precision.HIGHEST) + b3_ref[...]
    out_ref[...] = jnp.maximum(z, 0.0) + jnp.log1p(jnp.exp(-jnp.abs(z)))


def _head_call(p, g, x, degp, cb, w1, b1, w2, b2, w3, b3):
    full = lambda shape: pl.BlockSpec(shape, lambda j: tuple(0 for _ in shape))
    return pl.pallas_call(
        _head_body,
        grid=(_NB,),
        in_specs=[
            pl.BlockSpec((NC, _RB, C_IN), lambda j: (0, j, 0)),
            pl.BlockSpec((_RB, C_IN), lambda j: (j, 0)),
            pl.BlockSpec((_RB, C_IN), lambda j: (j, 0)),
            pl.BlockSpec((NC, _RB, 1), lambda j: (0, j, 0)),
            full((1, C_IN)),
            full((C_IN, HID)),
            full((1, HID)),
            full((HID, HID)),
            full((1, HID)),
            full((HID, 1)),
            full((1, 1)),
        ],
        out_specs=pl.BlockSpec((_RB, 1), lambda j: (j, 0)),
        out_shape=jax.ShapeDtypeStruct((N_NODES, 1), jnp.float32),
    )(p, g, x, degp, cb, w1, b1, w2, b2, w3, b3)


def _norm_body(c_ref, a_ref):
    c = c_ref[...]
    a_ref[...] = c / (jnp.sum(c) + 1e-5)


def _norm_call(conc):
    return pl.pallas_call(
        _norm_body,
        out_shape=jax.ShapeDtypeStruct((1, N_NODES), jnp.float32),
    )(conc)


def kernel(x, edge_index, conv_w, conv_b, lin1_w, lin1_b, lin2_w, lin2_b,
           lin3_w, lin3_b, deterministic):
    src = edge_index[0]
    dst = edge_index[1]
    ones128 = jnp.ones((128,), jnp.float32)
    zeros1 = jnp.zeros((NPAD,), jnp.float32)
    zerosw = jnp.zeros((NPAD, C_IN), jnp.float32)

    degp = (_deg_call(dst, ones128, zeros1)
            .reshape(NC, NPAD)[:, :N_NODES, None])
    g = _gscale_call(x, conv_w, degp)
    pflat = _agg_call(src, dst, g, zerosw)
    p = pflat.reshape(NC, NPAD, C_IN)[:, :N_NODES]
    conc = _head_call(p, g, x, degp, conv_b.reshape(1, C_IN),
                      lin1_w, lin1_b.reshape(1, HID),
                      lin2_w, lin2_b.reshape(1, HID),
                      lin3_w, lin3_b.reshape(1, 1))
    conc2 = conc.reshape(1, N_NODES)
    action = _norm_call(conc2)
    return action.reshape(N_NODES), conc2


# speedup vs baseline: 19.0403x; 1.1253x over previous
import functools

import jax
import jax.numpy as jnp
from jax import lax
from jax.experimental import pallas as pl
from jax.experimental.pallas import tpu as pltpu
from jax.experimental.pallas import tpu_sc as plsc

N_NODES = 10000
N_EDGES = 320000
EROWS = N_EDGES // 128
F = 64
C_IN = 128
HID = 256
NC, NS = 2, 16
NPAD = 10240
RPT = NPAD // NS


def _mesh():
    return plsc.VectorSubcoreMesh(core_axis_name="c", subcore_axis_name="s")


def _deg_body(dst_ref, ones_ref, zeros_ref, out_ref, idx_d, ones_v, acc):
    cid = lax.axis_index("c")
    sid = lax.axis_index("s")
    r0 = sid * RPT
    pltpu.sync_copy(zeros_ref.at[pl.ds(r0, RPT)], acc.at[pl.ds(r0, RPT)])
    pltpu.sync_copy(ones_ref, ones_v)
    plsc.subcore_barrier()
    half = EROWS // NC
    base = cid * half
    iters = (half + NS - 1) // NS

    def body(rr, carry):
        r = base + sid + rr * NS

        @pl.when(r < base + half)
        def _go():
            pltpu.sync_copy(dst_ref.at[pl.ds(r * 128, 128)], idx_d)
            pltpu.sync_copy(ones_v, acc.at[idx_d], add=True)

        return carry

    lax.fori_loop(0, iters, body, None)
    plsc.subcore_barrier()
    pltpu.sync_copy(acc.at[pl.ds(r0, RPT)],
                    out_ref.at[pl.ds(cid * NPAD + r0, RPT)])


_deg_call = pl.kernel(
    _deg_body,
    out_type=jax.ShapeDtypeStruct((NC * NPAD,), jnp.float32),
    mesh=_mesh(),
    scratch_types=[
        pltpu.VMEM((128,), jnp.int32),
        pltpu.VMEM((128,), jnp.float32),
        pltpu.VMEM_SHARED((NPAD,), jnp.float32),
    ],
)


_K = 2


def _agg_body(src_ref, dst_ref, g_ref, zeros_ref, out_ref,
              idx_s0, idx_s1, idx_d0, idx_d1,
              rows0, rows1, acc, sem):
    idx_s = [idx_s0, idx_s1]
    idx_d = [idx_d0, idx_d1]
    rows = [rows0, rows1]
    cid = lax.axis_index("c")
    sid = lax.axis_index("s")
    r0 = sid * RPT
    pltpu.sync_copy(zeros_ref.at[pl.ds(r0, RPT)], acc.at[pl.ds(r0, RPT)])
    plsc.subcore_barrier()
    half = EROWS // NC
    base = cid * half
    iters = (half + NS * _K - 1) // (NS * _K)

    def body(rr, carry):
        b0 = base + (sid + rr * NS) * _K
        for j in range(_K):
            @pl.when(b0 + j < base + half)
            def _start(j=j):
                pltpu.sync_copy(src_ref.at[pl.ds((b0 + j) * 128, 128)],
                                idx_s[j])
                pltpu.sync_copy(dst_ref.at[pl.ds((b0 + j) * 128, 128)],
                                idx_d[j])
                pltpu.make_async_copy(g_ref.at[idx_s[j]], rows[j],
                                      sem).start()
        for j in range(_K):
            @pl.when(b0 + j < base + half)
            def _drain(j=j):
                pltpu.make_async_copy(g_ref.at[idx_s[j]], rows[j],
                                      sem).wait()
        for j in range(_K):
            @pl.when(b0 + j < base + half)
            def _scat(j=j):
                pltpu.sync_copy(rows[j], acc.at[idx_d[j]], add=True)
        return carry

    lax.fori_loop(0, iters, body, None)
    plsc.subcore_barrier()
    pltpu.sync_copy(acc.at[pl.ds(r0, RPT)],
                    out_ref.at[pl.ds(cid * NPAD + r0, RPT)])


_agg_call = pl.kernel(
    _agg_body,
    out_type=jax.ShapeDtypeStruct((NC * NPAD, C_IN), jnp.float32),
    mesh=_mesh(),
    scratch_types=(
        [pltpu.VMEM((128,), jnp.int32)] * 4
        + [pltpu.VMEM((128, C_IN), jnp.float32)] * 2
        + [pltpu.VMEM_SHARED((NPAD, C_IN), jnp.float32),
           pltpu.SemaphoreType.DMA]
    ),
)


_RB = 1000
_NB = N_NODES // _RB


def _gscale_body(x_ref, w_ref, deg_ref, g_ref):
    h = jnp.dot(x_ref[...], w_ref[...], preferred_element_type=jnp.float32,
                precision=lax.Precision.HIGHEST)
    deg = deg_ref[0] + deg_ref[1] + 1.0
    g_ref[...] = h * lax.rsqrt(deg)


def _gscale_call(x, conv_w, degp):
    return pl.pallas_call(
        _gscale_body,
        grid=(_NB,),
        in_specs=[
            pl.BlockSpec((_RB, C_IN), lambda j: (j, 0)),
            pl.BlockSpec((C_IN, C_IN), lambda j: (0, 0)),
            pl.BlockSpec((NC, _RB, 1), lambda j: (0, j, 0)),
        ],
        out_specs=pl.BlockSpec((_RB, C_IN), lambda j: (j, 0)),
        out_shape=jax.ShapeDtypeStruct((N_NODES, C_IN), jnp.float32),
    )(x, conv_w, degp)


def _head_body(p_ref, g_ref, x_ref, deg_ref, cb_ref,
               w1_ref, b1_ref, w2_ref, b2_ref, w3_ref, b3_ref, out_ref):
    p = p_ref[0] + p_ref[1]
    g = g_ref[...]
    deg = deg_ref[0] + deg_ref[1] + 1.0
    dis = lax.rsqrt(deg)
    conv = (p + g) * dis + cb_ref[...]
    h2 = jnp.maximum(conv, 0.0) + x_ref[...]
    m1 = jnp.dot(h2, w1_ref[...], preferred_element_type=jnp.float32,
                 precision=lax.Precision.HIGHEST) + b1_ref[...]
    m1 = jnp.where(m1 > 0, m1, 0.01 * m1)
    m2 = jnp.dot(m1, w2_ref[...], preferred_element_type=jnp.float32,
                 precision=lax.Precision.HIGHEST) + b2_ref[...]
    m2 = jnp.where(m2 > 0, m2, 0.01 * m2)
    z = jnp.dot(m2, w3_ref[...], preferred_element_type=jnp.float32,
                precision=lax.Precision.HIGHEST) + b3_ref[...]
    out_ref[...] = jnp.maximum(z, 0.0) + jnp.log1p(jnp.exp(-jnp.abs(z)))


def _head_call(p, g, x, degp, cb, w1, b1, w2, b2, w3, b3):
    full = lambda shape: pl.BlockSpec(shape, lambda j: tuple(0 for _ in shape))
    return pl.pallas_call(
        _head_body,
        grid=(_NB,),
        in_specs=[
            pl.BlockSpec((NC, _RB, C_IN), lambda j: (0, j, 0)),
            pl.BlockSpec((_RB, C_IN), lambda j: (j, 0)),
            pl.BlockSpec((_RB, C_IN), lambda j: (j, 0)),
            pl.BlockSpec((NC, _RB, 1), lambda j: (0, j, 0)),
            full((1, C_IN)),
            full((C_IN, HID)),
            full((1, HID)),
            full((HID, HID)),
            full((1, HID)),
            full((HID, 1)),
            full((1, 1)),
        ],
        out_specs=pl.BlockSpec((_RB, 1), lambda j: (j, 0)),
        out_shape=jax.ShapeDtypeStruct((N_NODES, 1), jnp.float32),
    )(p, g, x, degp, cb, w1, b1, w2, b2, w3, b3)


def _norm_body(c_ref, a_ref):
    c = c_ref[...]
    a_ref[...] = c / (jnp.sum(c) + 1e-5)


def _norm_call(conc):
    return pl.pallas_call(
        _norm_body,
        out_shape=jax.ShapeDtypeStruct((1, N_NODES), jnp.float32),
    )(conc)


def kernel(x, edge_index, conv_w, conv_b, lin1_w, lin1_b, lin2_w, lin2_b,
           lin3_w, lin3_b, deterministic):
    src = edge_index[0]
    dst = edge_index[1]
    ones128 = jnp.ones((128,), jnp.float32)
    zeros1 = jnp.zeros((NPAD,), jnp.float32)
    zerosw = jnp.zeros((NPAD, C_IN), jnp.float32)

    degp = (_deg_call(dst, ones128, zeros1)
            .reshape(NC, NPAD)[:, :N_NODES, None])
    g = _gscale_call(x, conv_w, degp)
    pflat = _agg_call(src, dst, g, zerosw)
    p = pflat.reshape(NC, NPAD, C_IN)[:, :N_NODES]
    conc = _head_call(p, g, x, degp, conv_b.reshape(1, C_IN),
                      lin1_w, lin1_b.reshape(1, HID),
                      lin2_w, lin2_b.reshape(1, HID),
                      lin3_w, lin3_b.reshape(1, 1))
    conc2 = conc.reshape(1, N_NODES)
    action = _norm_call(conc2)
    return action.reshape(N_NODES), conc2


# speedup vs baseline: 20.5113x; 1.0773x over previous
import functools

import jax
import jax.numpy as jnp
from jax import lax
from jax.experimental import pallas as pl
from jax.experimental.pallas import tpu as pltpu
from jax.experimental.pallas import tpu_sc as plsc

N_NODES = 10000
N_EDGES = 320000
EROWS = N_EDGES // 128
F = 64
C_IN = 128
HID = 256
NC, NS = 2, 16
NPAD = 10240
RPT = NPAD // NS


def _mesh():
    return plsc.VectorSubcoreMesh(core_axis_name="c", subcore_axis_name="s")


def _deg_body(dst_ref, ones_ref, zeros_ref, out_ref, idx_d, ones_v, acc):
    cid = lax.axis_index("c")
    sid = lax.axis_index("s")
    r0 = sid * RPT
    pltpu.sync_copy(zeros_ref.at[pl.ds(r0, RPT)], acc.at[pl.ds(r0, RPT)])
    pltpu.sync_copy(ones_ref, ones_v)
    plsc.subcore_barrier()
    half = EROWS // NC
    base = cid * half
    iters = (half + NS - 1) // NS

    def body(rr, carry):
        r = base + sid + rr * NS

        @pl.when(r < base + half)
        def _go():
            pltpu.sync_copy(dst_ref.at[pl.ds(r * 128, 128)], idx_d)
            pltpu.sync_copy(ones_v, acc.at[idx_d], add=True)

        return carry

    lax.fori_loop(0, iters, body, None)
    plsc.subcore_barrier()
    pltpu.sync_copy(acc.at[pl.ds(r0, RPT)],
                    out_ref.at[pl.ds(cid * NPAD + r0, RPT)])


_deg_call = pl.kernel(
    _deg_body,
    out_type=jax.ShapeDtypeStruct((NC * NPAD,), jnp.float32),
    mesh=_mesh(),
    scratch_types=[
        pltpu.VMEM((128,), jnp.int32),
        pltpu.VMEM((128,), jnp.float32),
        pltpu.VMEM_SHARED((NPAD,), jnp.float32),
    ],
)


_K = 2


def _agg_body(src_ref, dst_ref, g_ref, zeros_ref, out_ref,
              idx_s0, idx_s1, idx_d0, idx_d1,
              rows0, rows1, acc, gsem, ssem):
    idx_s = [idx_s0, idx_s1]
    idx_d = [idx_d0, idx_d1]
    rows = [rows0, rows1]
    cid = lax.axis_index("c")
    sid = lax.axis_index("s")
    r0 = sid * RPT
    pltpu.sync_copy(zeros_ref.at[pl.ds(r0, RPT)], acc.at[pl.ds(r0, RPT)])
    plsc.subcore_barrier()
    half = EROWS // NC
    base = cid * half
    iters = (half + NS * _K - 1) // (NS * _K)

    def body(rr, carry):
        b0 = base + (sid + rr * NS) * _K
        prev = b0 - NS * _K
        for j in range(_K):
            @pl.when((prev >= base) & (prev + j < base + half))
            def _prev_drain(j=j):
                pltpu.make_async_copy(rows[j], acc.at[idx_d[j]],
                                      ssem).wait()
        for j in range(_K):
            @pl.when(b0 + j < base + half)
            def _start(j=j):
                pltpu.sync_copy(src_ref.at[pl.ds((b0 + j) * 128, 128)],
                                idx_s[j])
                pltpu.sync_copy(dst_ref.at[pl.ds((b0 + j) * 128, 128)],
                                idx_d[j])
                pltpu.make_async_copy(g_ref.at[idx_s[j]], rows[j],
                                      gsem).start()
        for j in range(_K):
            @pl.when(b0 + j < base + half)
            def _scat(j=j):
                pltpu.make_async_copy(g_ref.at[idx_s[j]], rows[j],
                                      gsem).wait()
                pltpu.async_copy(rows[j], acc.at[idx_d[j]], ssem, add=True)
        return carry

    lax.fori_loop(0, iters, body, None)
    lastb = base + (sid + (iters - 1) * NS) * _K
    for j in range(_K):
        @pl.when(lastb + j < base + half)
        def _final_drain(j=j):
            pltpu.make_async_copy(rows[j], acc.at[idx_d[j]], ssem).wait()
    plsc.subcore_barrier()
    pltpu.sync_copy(acc.at[pl.ds(r0, RPT)],
                    out_ref.at[pl.ds(cid * NPAD + r0, RPT)])


_agg_call = pl.kernel(
    _agg_body,
    out_type=jax.ShapeDtypeStruct((NC * NPAD, C_IN), jnp.float32),
    mesh=_mesh(),
    scratch_types=(
        [pltpu.VMEM((128,), jnp.int32)] * 4
        + [pltpu.VMEM((128, C_IN), jnp.float32)] * 2
        + [pltpu.VMEM_SHARED((NPAD, C_IN), jnp.float32),
           pltpu.SemaphoreType.DMA,
           pltpu.SemaphoreType.DMA]
    ),
)


_RB = 1000
_NB = N_NODES // _RB


def _gscale_body(x_ref, w_ref, deg_ref, g_ref):
    h = jnp.dot(x_ref[...], w_ref[...], preferred_element_type=jnp.float32,
                precision=lax.Precision.HIGHEST)
    deg = deg_ref[0] + deg_ref[1] + 1.0
    g_ref[...] = h * lax.rsqrt(deg)


def _gscale_call(x, conv_w, degp):
    return pl.pallas_call(
        _gscale_body,
        grid=(_NB,),
        in_specs=[
            pl.BlockSpec((_RB, C_IN), lambda j: (j, 0)),
            pl.BlockSpec((C_IN, C_IN), lambda j: (0, 0)),
            pl.BlockSpec((NC, _RB, 1), lambda j: (0, j, 0)),
        ],
        out_specs=pl.BlockSpec((_RB, C_IN), lambda j: (j, 0)),
        out_shape=jax.ShapeDtypeStruct((N_NODES, C_IN), jnp.float32),
    )(x, conv_w, degp)


def _head_body(p_ref, g_ref, x_ref, deg_ref, cb_ref,
               w1_ref, b1_ref, w2_ref, b2_ref, w3_ref, b3_ref, out_ref):
    p = p_ref[0] + p_ref[1]
    g = g_ref[...]
    deg = deg_ref[0] + deg_ref[1] + 1.0
    dis = lax.rsqrt(deg)
    conv = (p + g) * dis + cb_ref[...]
    h2 = jnp.maximum(conv, 0.0) + x_ref[...]
    m1 = jnp.dot(h2, w1_ref[...], preferred_element_type=jnp.float32,
                 precision=lax.Precision.HIGHEST) + b1_ref[...]
    m1 = jnp.where(m1 > 0, m1, 0.01 * m1)
    m2 = jnp.dot(m1, w2_ref[...], preferred_element_type=jnp.float32,
                 precision=lax.Precision.HIGHEST) + b2_ref[...]
    m2 = jnp.where(m2 > 0, m2, 0.01 * m2)
    z = jnp.dot(m2, w3_ref[...], preferred_element_type=jnp.float32,
                precision=lax.Precision.HIGHEST) + b3_ref[...]
    out_ref[...] = jnp.maximum(z, 0.0) + jnp.log1p(jnp.exp(-jnp.abs(z)))


def _head_call(p, g, x, degp, cb, w1, b1, w2, b2, w3, b3):
    full = lambda shape: pl.BlockSpec(shape, lambda j: tuple(0 for _ in shape))
    return pl.pallas_call(
        _head_body,
        grid=(_NB,),
        in_specs=[
            pl.BlockSpec((NC, _RB, C_IN), lambda j: (0, j, 0)),
            pl.BlockSpec((_RB, C_IN), lambda j: (j, 0)),
            pl.BlockSpec((_RB, C_IN), lambda j: (j, 0)),
            pl.BlockSpec((NC, _RB, 1), lambda j: (0, j, 0)),
            full((1, C_IN)),
            full((C_IN, HID)),
            full((1, HID)),
            full((HID, HID)),
            full((1, HID)),
            full((HID, 1)),
            full((1, 1)),
        ],
        out_specs=pl.BlockSpec((_RB, 1), lambda j: (j, 0)),
        out_shape=jax.ShapeDtypeStruct((N_NODES, 1), jnp.float32),
    )(p, g, x, degp, cb, w1, b1, w2, b2, w3, b3)


def _norm_body(c_ref, a_ref):
    c = c_ref[...]
    a_ref[...] = c / (jnp.sum(c) + 1e-5)


def _norm_call(conc):
    return pl.pallas_call(
        _norm_body,
        out_shape=jax.ShapeDtypeStruct((1, N_NODES), jnp.float32),
    )(conc)


def kernel(x, edge_index, conv_w, conv_b, lin1_w, lin1_b, lin2_w, lin2_b,
           lin3_w, lin3_b, deterministic):
    src = edge_index[0]
    dst = edge_index[1]
    ones128 = jnp.ones((128,), jnp.float32)
    zeros1 = jnp.zeros((NPAD,), jnp.float32)
    zerosw = jnp.zeros((NPAD, C_IN), jnp.float32)

    degp = _deg_call(dst, ones128, zeros1).reshape(NC, NPAD, 1)
    g = _gscale_call(x, conv_w, degp)
    pflat = _agg_call(src, dst, g, zerosw)
    p = pflat.reshape(NC, NPAD, C_IN)
    conc = _head_call(p, g, x, degp, conv_b.reshape(1, C_IN),
                      lin1_w, lin1_b.reshape(1, HID),
                      lin2_w, lin2_b.reshape(1, HID),
                      lin3_w, lin3_b.reshape(1, 1))
    conc2 = conc.reshape(1, N_NODES)
    action = _norm_call(conc2)
    return action.reshape(N_NODES), conc2


# speedup vs baseline: 22.2782x; 1.0861x over previous
import functools

import jax
import jax.numpy as jnp
from jax import lax
from jax.experimental import pallas as pl
from jax.experimental.pallas import tpu as pltpu
from jax.experimental.pallas import tpu_sc as plsc

N_NODES = 10000
N_EDGES = 320000
EROWS = N_EDGES // 128
F = 64
C_IN = 128
HID = 256
NC, NS = 2, 16
NPAD = 10240
RPT = NPAD // NS


def _mesh():
    return plsc.VectorSubcoreMesh(core_axis_name="c", subcore_axis_name="s")


def _deg_body(dst_ref, ones_ref, zeros_ref, out_ref, idx_d, ones_v, acc):
    cid = lax.axis_index("c")
    sid = lax.axis_index("s")
    r0 = sid * RPT
    pltpu.sync_copy(zeros_ref.at[pl.ds(r0, RPT)], acc.at[pl.ds(r0, RPT)])
    pltpu.sync_copy(ones_ref, ones_v)
    plsc.subcore_barrier()
    half = EROWS // NC
    base = cid * half
    iters = (half + NS - 1) // NS

    def body(rr, carry):
        r = base + sid + rr * NS

        @pl.when(r < base + half)
        def _go():
            pltpu.sync_copy(dst_ref.at[pl.ds(r * 128, 128)], idx_d)
            pltpu.sync_copy(ones_v, acc.at[idx_d], add=True)

        return carry

    lax.fori_loop(0, iters, body, None)
    plsc.subcore_barrier()
    pltpu.sync_copy(acc.at[pl.ds(r0, RPT)],
                    out_ref.at[pl.ds(cid * NPAD + r0, RPT)])


_deg_call = pl.kernel(
    _deg_body,
    out_type=jax.ShapeDtypeStruct((NC * NPAD,), jnp.float32),
    mesh=_mesh(),
    scratch_types=[
        pltpu.VMEM((128,), jnp.int32),
        pltpu.VMEM((128,), jnp.float32),
        pltpu.VMEM_SHARED((NPAD,), jnp.float32),
    ],
)


_K = 2


def _agg_body(src_ref, dst_ref, g_ref, zeros_ref, out_ref,
              idx_s0, idx_s1, idx_d0, idx_d1,
              rows0, rows1, acc, gsem, ssem):
    idx_s = [idx_s0, idx_s1]
    idx_d = [idx_d0, idx_d1]
    rows = [rows0, rows1]
    cid = lax.axis_index("c")
    sid = lax.axis_index("s")
    r0 = sid * RPT
    pltpu.sync_copy(zeros_ref.at[pl.ds(r0, RPT)], acc.at[pl.ds(r0, RPT)])
    plsc.subcore_barrier()
    half = EROWS // NC
    base = cid * half
    iters = (half + NS * _K - 1) // (NS * _K)

    def body(rr, carry):
        b0 = base + (sid + rr * NS) * _K
        prev = b0 - NS * _K
        for j in range(_K):
            @pl.when((prev >= base) & (prev + j < base + half))
            def _prev_drain(j=j):
                pltpu.make_async_copy(rows[j], acc.at[idx_d[j]],
                                      ssem).wait()
        for j in range(_K):
            @pl.when(b0 + j < base + half)
            def _start(j=j):
                pltpu.sync_copy(src_ref.at[pl.ds((b0 + j) * 128, 128)],
                                idx_s[j])
                pltpu.sync_copy(dst_ref.at[pl.ds((b0 + j) * 128, 128)],
                                idx_d[j])
                pltpu.make_async_copy(g_ref.at[idx_s[j]], rows[j],
                                      gsem).start()
        for j in range(_K):
            @pl.when(b0 + j < base + half)
            def _scat(j=j):
                pltpu.make_async_copy(g_ref.at[idx_s[j]], rows[j],
                                      gsem).wait()
                pltpu.async_copy(rows[j], acc.at[idx_d[j]], ssem, add=True)
        return carry

    lax.fori_loop(0, iters, body, None)
    lastb = base + (sid + (iters - 1) * NS) * _K
    for j in range(_K):
        @pl.when(lastb + j < base + half)
        def _final_drain(j=j):
            pltpu.make_async_copy(rows[j], acc.at[idx_d[j]], ssem).wait()
    plsc.subcore_barrier()
    pltpu.sync_copy(acc.at[pl.ds(r0, RPT)],
                    out_ref.at[pl.ds(cid * NPAD + r0, RPT)])


_agg_call = pl.kernel(
    _agg_body,
    out_type=jax.ShapeDtypeStruct((NC * NPAD, C_IN), jnp.float32),
    mesh=_mesh(),
    scratch_types=(
        [pltpu.VMEM((128,), jnp.int32)] * 4
        + [pltpu.VMEM((128, C_IN), jnp.float32)] * 2
        + [pltpu.VMEM_SHARED((NPAD, C_IN), jnp.float32),
           pltpu.SemaphoreType.DMA,
           pltpu.SemaphoreType.DMA]
    ),
)


_RB = 1000
_NB = N_NODES // _RB


def _gscale_body(x_ref, w_ref, deg_ref, g_ref):
    h = jnp.dot(x_ref[...], w_ref[...], preferred_element_type=jnp.float32,
                precision=lax.Precision.DEFAULT)
    deg = deg_ref[0] + deg_ref[1] + 1.0
    g_ref[...] = h * lax.rsqrt(deg)


def _gscale_call(x, conv_w, degp):
    return pl.pallas_call(
        _gscale_body,
        grid=(_NB,),
        in_specs=[
            pl.BlockSpec((_RB, C_IN), lambda j: (j, 0)),
            pl.BlockSpec((C_IN, C_IN), lambda j: (0, 0)),
            pl.BlockSpec((NC, _RB, 1), lambda j: (0, j, 0)),
        ],
        out_specs=pl.BlockSpec((_RB, C_IN), lambda j: (j, 0)),
        out_shape=jax.ShapeDtypeStruct((N_NODES, C_IN), jnp.float32),
    )(x, conv_w, degp)


def _head_body(p_ref, g_ref, x_ref, deg_ref, cb_ref,
               w1_ref, b1_ref, w2_ref, b2_ref, w3_ref, b3_ref,
               conc_ref, act_ref):
    j = pl.program_id(0)
    p = p_ref[0] + p_ref[1]
    g = g_ref[...]
    deg = deg_ref[0] + deg_ref[1] + 1.0
    dis = lax.rsqrt(deg)
    conv = (p + g) * dis + cb_ref[...]
    h2 = jnp.maximum(conv, 0.0) + x_ref[...]
    m1 = jnp.dot(h2, w1_ref[...], preferred_element_type=jnp.float32,
                 precision=lax.Precision.DEFAULT) + b1_ref[...]
    m1 = jnp.where(m1 > 0, m1, 0.01 * m1)
    m2 = jnp.dot(m1, w2_ref[...], preferred_element_type=jnp.float32,
                 precision=lax.Precision.DEFAULT) + b2_ref[...]
    m2 = jnp.where(m2 > 0, m2, 0.01 * m2)
    z = jnp.dot(m2, w3_ref[...], preferred_element_type=jnp.float32,
                precision=lax.Precision.DEFAULT) + b3_ref[...]
    sp = jnp.maximum(z, 0.0) + jnp.log1p(jnp.exp(-jnp.abs(z)))
    conc_ref[...] = sp
    act_ref[pl.ds(j * _RB, _RB), :] = sp

    @pl.when(j == _NB - 1)
    def _norm():
        act_ref[...] = act_ref[...] / (jnp.sum(act_ref[...]) + 1e-5)


def _head_call(p, g, x, degp, cb, w1, b1, w2, b2, w3, b3):
    full = lambda shape: pl.BlockSpec(shape, lambda j: tuple(0 for _ in shape))
    return pl.pallas_call(
        _head_body,
        grid=(_NB,),
        in_specs=[
            pl.BlockSpec((NC, _RB, C_IN), lambda j: (0, j, 0)),
            pl.BlockSpec((_RB, C_IN), lambda j: (j, 0)),
            pl.BlockSpec((_RB, C_IN), lambda j: (j, 0)),
            pl.BlockSpec((NC, _RB, 1), lambda j: (0, j, 0)),
            full((1, C_IN)),
            full((C_IN, HID)),
            full((1, HID)),
            full((HID, HID)),
            full((1, HID)),
            full((HID, 1)),
            full((1, 1)),
        ],
        out_specs=[
            pl.BlockSpec((_RB, 1), lambda j: (j, 0)),
            pl.BlockSpec((N_NODES, 1), lambda j: (0, 0)),
        ],
        out_shape=[
            jax.ShapeDtypeStruct((N_NODES, 1), jnp.float32),
            jax.ShapeDtypeStruct((N_NODES, 1), jnp.float32),
        ],
    )(p, g, x, degp, cb, w1, b1, w2, b2, w3, b3)


def kernel(x, edge_index, conv_w, conv_b, lin1_w, lin1_b, lin2_w, lin2_b,
           lin3_w, lin3_b, deterministic):
    src = edge_index[0]
    dst = edge_index[1]
    ones128 = jnp.ones((128,), jnp.float32)
    zeros1 = jnp.zeros((NPAD,), jnp.float32)
    zerosw = jnp.zeros((NPAD, C_IN), jnp.float32)

    degp = _deg_call(dst, ones128, zeros1).reshape(NC, NPAD, 1)
    g = _gscale_call(x, conv_w, degp)
    pflat = _agg_call(src, dst, g, zerosw)
    p = pflat.reshape(NC, NPAD, C_IN)
    conc, act = _head_call(p, g, x, degp, conv_b.reshape(1, C_IN),
                           lin1_w, lin1_b.reshape(1, HID),
                           lin2_w, lin2_b.reshape(1, HID),
                           lin3_w, lin3_b.reshape(1, 1))
    return act.reshape(N_NODES), conc.reshape(1, N_NODES)


# speedup vs baseline: 25.1714x; 1.1299x over previous
import functools

import jax
import jax.numpy as jnp
from jax import lax
from jax.experimental import pallas as pl
from jax.experimental.pallas import tpu as pltpu
from jax.experimental.pallas import tpu_sc as plsc

N_NODES = 10000
N_EDGES = 320000
EROWS = N_EDGES // 128
F = 64
C_IN = 128
HID = 256
NC, NS = 2, 16
NPAD = 10240
RPT = NPAD // NS


def _mesh():
    return plsc.VectorSubcoreMesh(core_axis_name="c", subcore_axis_name="s")


_DCHUNK = 1280
_DCH_PER_CORE = (N_EDGES // NC) // _DCHUNK


def _deg_body(dst_ref, ones_ref, zeros_ref, out_ref, idx_d, ones_v, acc):
    cid = lax.axis_index("c")
    sid = lax.axis_index("s")
    r0 = sid * RPT
    pltpu.sync_copy(zeros_ref.at[pl.ds(r0, RPT)], acc.at[pl.ds(r0, RPT)])
    pltpu.sync_copy(ones_ref, ones_v)
    plsc.subcore_barrier()
    base = cid * (N_EDGES // NC)
    iters = (_DCH_PER_CORE + NS - 1) // NS

    def body(rr, carry):
        c = sid + rr * NS

        @pl.when(c < _DCH_PER_CORE)
        def _go():
            pltpu.sync_copy(dst_ref.at[pl.ds(base + c * _DCHUNK, _DCHUNK)],
                            idx_d)
            pltpu.sync_copy(ones_v, acc.at[idx_d], add=True)

        return carry

    lax.fori_loop(0, iters, body, None)
    plsc.subcore_barrier()
    pltpu.sync_copy(acc.at[pl.ds(r0, RPT)],
                    out_ref.at[pl.ds(cid * NPAD + r0, RPT)])


_deg_call = pl.kernel(
    _deg_body,
    out_type=jax.ShapeDtypeStruct((NC * NPAD,), jnp.float32),
    mesh=_mesh(),
    scratch_types=[
        pltpu.VMEM((_DCHUNK,), jnp.int32),
        pltpu.VMEM((_DCHUNK,), jnp.float32),
        pltpu.VMEM_SHARED((NPAD,), jnp.float32),
    ],
)


_K = 2


def _agg_body(src_ref, dst_ref, g_ref, zeros_ref, out_ref,
              idx_s0, idx_s1, idx_d0, idx_d1,
              rows0, rows1, acc, gsem, ssem):
    idx_s = [idx_s0, idx_s1]
    idx_d = [idx_d0, idx_d1]
    rows = [rows0, rows1]
    cid = lax.axis_index("c")
    sid = lax.axis_index("s")
    r0 = sid * RPT
    pltpu.sync_copy(zeros_ref.at[pl.ds(r0, RPT)], acc.at[pl.ds(r0, RPT)])
    plsc.subcore_barrier()
    half = EROWS // NC
    base = cid * half
    iters = (half + NS * _K - 1) // (NS * _K)

    def body(rr, carry):
        b0 = base + (sid + rr * NS) * _K
        prev = b0 - NS * _K
        for j in range(_K):
            @pl.when((prev >= base) & (prev + j < base + half))
            def _prev_drain(j=j):
                pltpu.make_async_copy(rows[j], acc.at[idx_d[j]],
                                      ssem).wait()
        for j in range(_K):
            @pl.when(b0 + j < base + half)
            def _start(j=j):
                pltpu.sync_copy(src_ref.at[pl.ds((b0 + j) * 128, 128)],
                                idx_s[j])
                pltpu.sync_copy(dst_ref.at[pl.ds((b0 + j) * 128, 128)],
                                idx_d[j])
                pltpu.make_async_copy(g_ref.at[idx_s[j]], rows[j],
                                      gsem).start()
        for j in range(_K):
            @pl.when(b0 + j < base + half)
            def _scat(j=j):
                pltpu.make_async_copy(g_ref.at[idx_s[j]], rows[j],
                                      gsem).wait()
                pltpu.async_copy(rows[j], acc.at[idx_d[j]], ssem, add=True)
        return carry

    lax.fori_loop(0, iters, body, None)
    lastb = base + (sid + (iters - 1) * NS) * _K
    for j in range(_K):
        @pl.when(lastb + j < base + half)
        def _final_drain(j=j):
            pltpu.make_async_copy(rows[j], acc.at[idx_d[j]], ssem).wait()
    plsc.subcore_barrier()
    pltpu.sync_copy(acc.at[pl.ds(r0, RPT)],
                    out_ref.at[pl.ds(cid * NPAD + r0, RPT)])


_agg_call = pl.kernel(
    _agg_body,
    out_type=jax.ShapeDtypeStruct((NC * NPAD, C_IN), jnp.float32),
    mesh=_mesh(),
    scratch_types=(
        [pltpu.VMEM((128,), jnp.int32)] * 4
        + [pltpu.VMEM((128, C_IN), jnp.float32)] * 2
        + [pltpu.VMEM_SHARED((NPAD, C_IN), jnp.float32),
           pltpu.SemaphoreType.DMA,
           pltpu.SemaphoreType.DMA]
    ),
)


_RB = 1000
_NB = N_NODES // _RB


def _gscale_body(x_ref, w_ref, deg_ref, g_ref):
    h = jnp.dot(x_ref[...], w_ref[...], preferred_element_type=jnp.float32,
                precision=lax.Precision.DEFAULT)
    deg = deg_ref[0] + deg_ref[1] + 1.0
    g_ref[...] = h * lax.rsqrt(deg)


def _gscale_call(x, conv_w, degp):
    return pl.pallas_call(
        _gscale_body,
        grid=(_NB,),
        in_specs=[
            pl.BlockSpec((_RB, C_IN), lambda j: (j, 0)),
            pl.BlockSpec((C_IN, C_IN), lambda j: (0, 0)),
            pl.BlockSpec((NC, _RB, 1), lambda j: (0, j, 0)),
        ],
        out_specs=pl.BlockSpec((_RB, C_IN), lambda j: (j, 0)),
        out_shape=jax.ShapeDtypeStruct((N_NODES, C_IN), jnp.float32),
    )(x, conv_w, degp)


def _head_body(p_ref, g_ref, x_ref, deg_ref, cb_ref,
               w1_ref, b1_ref, w2_ref, b2_ref, w3_ref, b3_ref,
               conc_ref, act_ref):
    j = pl.program_id(0)
    p = p_ref[0] + p_ref[1]
    g = g_ref[...]
    deg = deg_ref[0] + deg_ref[1] + 1.0
    dis = lax.rsqrt(deg)
    conv = (p + g) * dis + cb_ref[...]
    h2 = jnp.maximum(conv, 0.0) + x_ref[...]
    m1 = jnp.dot(h2, w1_ref[...], preferred_element_type=jnp.float32,
                 precision=lax.Precision.DEFAULT) + b1_ref[...]
    m1 = jnp.where(m1 > 0, m1, 0.01 * m1)
    m2 = jnp.dot(m1, w2_ref[...], preferred_element_type=jnp.float32,
                 precision=lax.Precision.DEFAULT) + b2_ref[...]
    m2 = jnp.where(m2 > 0, m2, 0.01 * m2)
    z = jnp.dot(m2, w3_ref[...], preferred_element_type=jnp.float32,
                precision=lax.Precision.DEFAULT) + b3_ref[...]
    sp = jnp.maximum(z, 0.0) + jnp.log1p(jnp.exp(-jnp.abs(z)))
    conc_ref[...] = sp
    act_ref[pl.ds(j * _RB, _RB), :] = sp

    @pl.when(j == _NB - 1)
    def _norm():
        act_ref[...] = act_ref[...] / (jnp.sum(act_ref[...]) + 1e-5)


def _head_call(p, g, x, degp, cb, w1, b1, w2, b2, w3, b3):
    full = lambda shape: pl.BlockSpec(shape, lambda j: tuple(0 for _ in shape))
    return pl.pallas_call(
        _head_body,
        grid=(_NB,),
        in_specs=[
            pl.BlockSpec((NC, _RB, C_IN), lambda j: (0, j, 0)),
            pl.BlockSpec((_RB, C_IN), lambda j: (j, 0)),
            pl.BlockSpec((_RB, C_IN), lambda j: (j, 0)),
            pl.BlockSpec((NC, _RB, 1), lambda j: (0, j, 0)),
            full((1, C_IN)),
            full((C_IN, HID)),
            full((1, HID)),
            full((HID, HID)),
            full((1, HID)),
            full((HID, 1)),
            full((1, 1)),
        ],
        out_specs=[
            pl.BlockSpec((_RB, 1), lambda j: (j, 0)),
            pl.BlockSpec((N_NODES, 1), lambda j: (0, 0)),
        ],
        out_shape=[
            jax.ShapeDtypeStruct((N_NODES, 1), jnp.float32),
            jax.ShapeDtypeStruct((N_NODES, 1), jnp.float32),
        ],
    )(p, g, x, degp, cb, w1, b1, w2, b2, w3, b3)


def kernel(x, edge_index, conv_w, conv_b, lin1_w, lin1_b, lin2_w, lin2_b,
           lin3_w, lin3_b, deterministic):
    src = edge_index[0]
    dst = edge_index[1]
    ones128 = jnp.ones((_DCHUNK,), jnp.float32)
    zeros1 = jnp.zeros((NPAD,), jnp.float32)
    zerosw = jnp.zeros((NPAD, C_IN), jnp.float32)

    degp = _deg_call(dst, ones128, zeros1).reshape(NC, NPAD, 1)
    g = _gscale_call(x, conv_w, degp)
    pflat = _agg_call(src, dst, g, zerosw)
    p = pflat.reshape(NC, NPAD, C_IN)
    conc, act = _head_call(p, g, x, degp, conv_b.reshape(1, C_IN),
                           lin1_w, lin1_b.reshape(1, HID),
                           lin2_w, lin2_b.reshape(1, HID),
                           lin3_w, lin3_b.reshape(1, 1))
    return act.reshape(N_NODES), conc.reshape(1, N_NODES)


# speedup vs baseline: 26.1086x; 1.0372x over previous
import functools

import jax
import jax.numpy as jnp
from jax import lax
from jax.experimental import pallas as pl
from jax.experimental.pallas import tpu as pltpu
from jax.experimental.pallas import tpu_sc as plsc

N_NODES = 10000
N_EDGES = 320000
EROWS = N_EDGES // 128
F = 64
C_IN = 128
HID = 256
NC, NS = 2, 16
NPAD = 10240
RPT = NPAD // NS


def _mesh():
    return plsc.VectorSubcoreMesh(core_axis_name="c", subcore_axis_name="s")


_DCHUNK = 1280
_DCH_PER_CORE = (N_EDGES // NC) // _DCHUNK


def _deg_body(dst_ref, ones_ref, zeros_ref, out_ref, idx_d, ones_v, acc):
    cid = lax.axis_index("c")
    sid = lax.axis_index("s")
    r0 = sid * RPT
    pltpu.sync_copy(zeros_ref.at[pl.ds(r0, RPT)], acc.at[pl.ds(r0, RPT)])
    pltpu.sync_copy(ones_ref, ones_v)
    plsc.subcore_barrier()
    base = cid * (N_EDGES // NC)
    iters = (_DCH_PER_CORE + NS - 1) // NS

    def body(rr, carry):
        c = sid + rr * NS

        @pl.when(c < _DCH_PER_CORE)
        def _go():
            pltpu.sync_copy(dst_ref.at[pl.ds(base + c * _DCHUNK, _DCHUNK)],
                            idx_d)
            pltpu.sync_copy(ones_v, acc.at[idx_d], add=True)

        return carry

    lax.fori_loop(0, iters, body, None)
    plsc.subcore_barrier()
    pltpu.sync_copy(acc.at[pl.ds(r0, RPT)],
                    out_ref.at[pl.ds(cid * NPAD + r0, RPT)])


_deg_call = pl.kernel(
    _deg_body,
    out_type=jax.ShapeDtypeStruct((NC * NPAD,), jnp.float32),
    mesh=_mesh(),
    scratch_types=[
        pltpu.VMEM((_DCHUNK,), jnp.int32),
        pltpu.VMEM((_DCHUNK,), jnp.float32),
        pltpu.VMEM_SHARED((NPAD,), jnp.float32),
    ],
)


_K = 2


def _agg_body(src_ref, dst_ref, g_ref, zeros_ref, out_ref,
              idx_sb, idx_db, rows0, rows1, acc, gsem, ssem):
    idx_s = [idx_sb.at[pl.ds(0, 128)], idx_sb.at[pl.ds(128, 128)]]
    idx_d = [idx_db.at[pl.ds(0, 128)], idx_db.at[pl.ds(128, 128)]]
    rows = [rows0, rows1]
    cid = lax.axis_index("c")
    sid = lax.axis_index("s")
    r0 = sid * RPT
    pltpu.sync_copy(zeros_ref.at[pl.ds(r0, RPT)], acc.at[pl.ds(r0, RPT)])
    plsc.subcore_barrier()
    half = EROWS // NC
    base = cid * half
    iters = (half + NS * _K - 1) // (NS * _K)

    def body(rr, carry):
        b0 = base + (sid + rr * NS) * _K
        prev = b0 - NS * _K
        for j in range(_K):
            @pl.when((prev >= base) & (prev + j < base + half))
            def _prev_drain(j=j):
                pltpu.make_async_copy(rows[j], acc.at[idx_d[j]],
                                      ssem).wait()
        @pl.when(b0 < base + half)
        def _load():
            pltpu.sync_copy(src_ref.at[pl.ds(b0 * 128, _K * 128)], idx_sb)
            pltpu.sync_copy(dst_ref.at[pl.ds(b0 * 128, _K * 128)], idx_db)

        for j in range(_K):
            @pl.when(b0 + j < base + half)
            def _start(j=j):
                pltpu.make_async_copy(g_ref.at[idx_s[j]], rows[j],
                                      gsem).start()
        for j in range(_K):
            @pl.when(b0 + j < base + half)
            def _scat(j=j):
                pltpu.make_async_copy(g_ref.at[idx_s[j]], rows[j],
                                      gsem).wait()
                pltpu.async_copy(rows[j], acc.at[idx_d[j]], ssem, add=True)
        return carry

    lax.fori_loop(0, iters, body, None)
    lastb = base + (sid + (iters - 1) * NS) * _K
    for j in range(_K):
        @pl.when(lastb + j < base + half)
        def _final_drain(j=j):
            pltpu.make_async_copy(rows[j], acc.at[idx_d[j]], ssem).wait()
    plsc.subcore_barrier()
    pltpu.sync_copy(acc.at[pl.ds(r0, RPT)],
                    out_ref.at[pl.ds(cid * NPAD + r0, RPT)])


_agg_call = pl.kernel(
    _agg_body,
    out_type=jax.ShapeDtypeStruct((NC * NPAD, C_IN), jnp.float32),
    mesh=_mesh(),
    scratch_types=(
        [pltpu.VMEM((_K * 128,), jnp.int32)] * 2
        + [pltpu.VMEM((128, C_IN), jnp.float32)] * 2
        + [pltpu.VMEM_SHARED((NPAD, C_IN), jnp.float32),
           pltpu.SemaphoreType.DMA,
           pltpu.SemaphoreType.DMA]
    ),
)


_RB = 1000
_NB = N_NODES // _RB


def _gscale_body(x_ref, w_ref, deg_ref, g_ref):
    h = jnp.dot(x_ref[...], w_ref[...], preferred_element_type=jnp.float32,
                precision=lax.Precision.DEFAULT)
    deg = deg_ref[0] + deg_ref[1] + 1.0
    g_ref[...] = h * lax.rsqrt(deg)


def _gscale_call(x, conv_w, degp):
    return pl.pallas_call(
        _gscale_body,
        grid=(_NB,),
        in_specs=[
            pl.BlockSpec((_RB, C_IN), lambda j: (j, 0)),
            pl.BlockSpec((C_IN, C_IN), lambda j: (0, 0)),
            pl.BlockSpec((NC, _RB, 1), lambda j: (0, j, 0)),
        ],
        out_specs=pl.BlockSpec((_RB, C_IN), lambda j: (j, 0)),
        out_shape=jax.ShapeDtypeStruct((N_NODES, C_IN), jnp.float32),
    )(x, conv_w, degp)


def _head_body(p_ref, g_ref, x_ref, deg_ref, cb_ref,
               w1_ref, b1_ref, w2_ref, b2_ref, w3_ref, b3_ref,
               conc_ref, act_ref):
    j = pl.program_id(0)
    p = p_ref[0] + p_ref[1]
    g = g_ref[...]
    deg = deg_ref[0] + deg_ref[1] + 1.0
    dis = lax.rsqrt(deg)
    conv = (p + g) * dis + cb_ref[...]
    h2 = jnp.maximum(conv, 0.0) + x_ref[...]
    m1 = jnp.dot(h2, w1_ref[...], preferred_element_type=jnp.float32,
                 precision=lax.Precision.DEFAULT) + b1_ref[...]
    m1 = jnp.where(m1 > 0, m1, 0.01 * m1)
    m2 = jnp.dot(m1, w2_ref[...], preferred_element_type=jnp.float32,
                 precision=lax.Precision.DEFAULT) + b2_ref[...]
    m2 = jnp.where(m2 > 0, m2, 0.01 * m2)
    z = jnp.dot(m2, w3_ref[...], preferred_element_type=jnp.float32,
                precision=lax.Precision.DEFAULT) + b3_ref[...]
    sp = jnp.maximum(z, 0.0) + jnp.log1p(jnp.exp(-jnp.abs(z)))
    conc_ref[...] = sp
    act_ref[pl.ds(j * _RB, _RB), :] = sp

    @pl.when(j == _NB - 1)
    def _norm():
        act_ref[...] = act_ref[...] / (jnp.sum(act_ref[...]) + 1e-5)


def _head_call(p, g, x, degp, cb, w1, b1, w2, b2, w3, b3):
    full = lambda shape: pl.BlockSpec(shape, lambda j: tuple(0 for _ in shape))
    return pl.pallas_call(
        _head_body,
        grid=(_NB,),
        in_specs=[
            pl.BlockSpec((NC, _RB, C_IN), lambda j: (0, j, 0)),
            pl.BlockSpec((_RB, C_IN), lambda j: (j, 0)),
            pl.BlockSpec((_RB, C_IN), lambda j: (j, 0)),
            pl.BlockSpec((NC, _RB, 1), lambda j: (0, j, 0)),
            full((1, C_IN)),
            full((C_IN, HID)),
            full((1, HID)),
            full((HID, HID)),
            full((1, HID)),
            full((HID, 1)),
            full((1, 1)),
        ],
        out_specs=[
            pl.BlockSpec((_RB, 1), lambda j: (j, 0)),
            pl.BlockSpec((N_NODES, 1), lambda j: (0, 0)),
        ],
        out_shape=[
            jax.ShapeDtypeStruct((N_NODES, 1), jnp.float32),
            jax.ShapeDtypeStruct((N_NODES, 1), jnp.float32),
        ],
    )(p, g, x, degp, cb, w1, b1, w2, b2, w3, b3)


def kernel(x, edge_index, conv_w, conv_b, lin1_w, lin1_b, lin2_w, lin2_b,
           lin3_w, lin3_b, deterministic):
    src = edge_index[0]
    dst = edge_index[1]
    ones128 = jnp.ones((_DCHUNK,), jnp.float32)
    zeros1 = jnp.zeros((NPAD,), jnp.float32)
    zerosw = jnp.zeros((NPAD, C_IN), jnp.float32)

    degp = _deg_call(dst, ones128, zeros1).reshape(NC, NPAD, 1)
    g = _gscale_call(x, conv_w, degp)
    pflat = _agg_call(src, dst, g, zerosw)
    p = pflat.reshape(NC, NPAD, C_IN)
    conc, act = _head_call(p, g, x, degp, conv_b.reshape(1, C_IN),
                           lin1_w, lin1_b.reshape(1, HID),
                           lin2_w, lin2_b.reshape(1, HID),
                           lin3_w, lin3_b.reshape(1, 1))
    return act.reshape(N_NODES), conc.reshape(1, N_NODES)
